# baseline (speedup 1.0000x reference)
import jax
import jax.numpy as jnp
from jax import lax
from jax.experimental import pallas as pl
from jax.experimental.pallas import tpu as pltpu
from jax.experimental.pallas import tpu_sc as plsc

_ALPHA = 0.2
_BASE_MARGIN = 1.0
_ADAPT_RATE = 0.3

_N = 16384
_C = 1000
_F = 128
_B = 2048
_NB = _N // _B

_NC = 2
_NS = 16
_L = 16
_NW = _NC * _NS
_RW = _N // _NW


def _feat_body(feat_hbm, out_hbm, rows_v, acc_v, sem):
    w = lax.axis_index("s") * _NC + lax.axis_index("c")
    base = w * _RW
    pltpu.sync_copy(feat_hbm.at[pl.ds(base, _RW)], rows_v)
    lane = lax.broadcasted_iota(jnp.int32, (_L,), 0)

    def group(g, carry):
        out = jnp.zeros((_L,), jnp.float32)
        for jj in range(_L):
            r = g * _L + jj
            acc = jnp.zeros((_L,), jnp.float32)
            for j in range(_F // _L):
                v = rows_v[r, pl.ds(j * _L, _L)]
                acc = acc + v * v
            for sh in (8, 4, 2, 1):
                idx = (lane + sh) & (_L - 1)
                acc = acc + jnp.take(acc, idx)
            out = jnp.where(lane == jj, acc, out)
        acc_v[pl.ds(g * _L, _L)] = out
        return carry

    lax.fori_loop(0, _RW // _L, group, 0)
    pltpu.sync_copy(acc_v, out_hbm.at[pl.ds(base, _RW)])


def _feat_sumsq(features):
    mesh = plsc.VectorSubcoreMesh(core_axis_name="c", subcore_axis_name="s")
    return pl.kernel(
        _feat_body,
        mesh=mesh,
        out_type=jax.ShapeDtypeStruct((_N,), jnp.float32),
        scratch_types=[
            pltpu.VMEM((_RW, _F), jnp.float32),
            pltpu.VMEM((_RW,), jnp.float32),
            pltpu.SemaphoreType.DMA,
        ],
    )(features)


def _body(xt_ref, tgt_ref, fsq_ref, out_ref, loss_s, unc_s):
    i = pl.program_id(0)

    @pl.when(i < _NB)
    def dense_step():
        x = xt_ref[...]
        t = tgt_ref[...]

        colmax = jnp.max(x, axis=0, keepdims=True)
        s = jnp.sum(jnp.exp(x - colmax), axis=0, keepdims=True)
        logs = jnp.log(s)
        rows = jax.lax.broadcasted_iota(jnp.int32, (_C, _B), 0)
        tl = jnp.sum(jnp.where(rows == t, x, 0.0), axis=0, keepdims=True)
        loss_s[pl.ds(i, 1), :] = (colmax + logs) - tl
        unc_s[pl.ds(i, 1), :] = 1.0 - 1.0 / s

    @pl.when(i == _NB)
    def select_step():
        l = loss_s[...]
        u = unc_s[...]
        fn = jnp.sqrt(fsq_ref[...])
        nf = jnp.float32(_N)
        mean = jnp.sum(l) / nf
        var = jnp.sum((l - mean) ** 2) / (nf - 1.0)
        std = jnp.sqrt(var)
        alpha = jnp.clip(_ALPHA * (1.0 + std), 0.05, 0.5)
        k = jnp.maximum(1, jnp.ceil(nf * alpha)).astype(jnp.int32)

        bits = jax.lax.bitcast_convert_type(l, jnp.int32)
        key = jnp.where(bits < 0, bits ^ jnp.int32(0x7FFFFFFF), bits)
        min32 = jnp.int32(-2147483648)

        def body_tau(j, t):
            t2 = t | (jnp.int32(1) << (jnp.int32(31) - j))
            c = jnp.sum((key >= (t2 ^ min32)).astype(jnp.int32))
            return jnp.where(c >= k, t2, t)

        tau_u = jax.lax.fori_loop(0, 32, body_tau, jnp.int32(0))
        tau = tau_u ^ min32

        c_gt = jnp.sum((key > tau).astype(jnp.int32))
        m = k - c_gt
        tied = key == tau
        ii = (jax.lax.broadcasted_iota(jnp.int32, (_NB, _B), 0) * _B
              + jax.lax.broadcasted_iota(jnp.int32, (_NB, _B), 1))

        def body_idx(j, t):
            t2 = t | (jnp.int32(1) << (jnp.int32(13) - j))
            c = jnp.sum((tied & (ii < t2)).astype(jnp.int32))
            return jnp.where(c < m, t2, t)

        t_idx = jax.lax.fori_loop(0, 14, body_idx, jnp.int32(0))

        include = (key > tau) | (tied & (ii <= t_idx))
        contrib = l * (_BASE_MARGIN * (1.0 + _ADAPT_RATE * u)) + 0.1 * fn
        total = jnp.sum(jnp.where(include, contrib, 0.0))
        out_ref[...] = (total / k.astype(jnp.float32)).reshape(1, 1)


def kernel(outputs, targets, features):
    xt = outputs.T
    tgt2 = targets.reshape(1, _N)
    fsq = _feat_sumsq(features).reshape(_NB, _B)
    last = _NB - 1
    out = pl.pallas_call(
        _body,
        grid=(_NB + 1,),
        in_specs=[
            pl.BlockSpec((_C, _B), lambda i: (0, jnp.minimum(i, last))),
            pl.BlockSpec((1, _B), lambda i: (0, jnp.minimum(i, last))),
            pl.BlockSpec((_NB, _B), lambda i: (0, 0)),
        ],
        out_specs=pl.BlockSpec((1, 1), lambda i: (0, 0)),
        out_shape=jax.ShapeDtypeStruct((1, 1), jnp.float32),
        scratch_shapes=[pltpu.VMEM((_NB, _B), jnp.float32)] * 2,
    )(xt, tgt2, fsq)
    return out[0, 0]


# speedup vs baseline: 1.1725x; 1.1725x over previous
import jax
import jax.numpy as jnp
from jax import lax
from jax.experimental import pallas as pl
from jax.experimental.pallas import tpu as pltpu
from jax.experimental.pallas import tpu_sc as plsc

_ALPHA = 0.2
_BASE_MARGIN = 1.0
_ADAPT_RATE = 0.3

_N = 16384
_C = 1000
_F = 128
_B = 2048
_NB = _N // _B
_R2 = 128

_NC = 2
_NS = 16
_L = 16
_NW = _NC * _NS
_RW = _N // _NW


def _feat_body(feat_hbm, out_hbm, rows_v, acc_v, sem):
    w = lax.axis_index("s") * _NC + lax.axis_index("c")
    base = w * _RW
    pltpu.sync_copy(feat_hbm.at[pl.ds(base, _RW)], rows_v)
    lane = lax.broadcasted_iota(jnp.int32, (_L,), 0)

    def group(g, carry):
        out = jnp.zeros((_L,), jnp.float32)
        for jj in range(_L):
            r = g * _L + jj
            acc = jnp.zeros((_L,), jnp.float32)
            for j in range(_F // _L):
                v = rows_v[r, pl.ds(j * _L, _L)]
                acc = acc + v * v
            for sh in (8, 4, 2, 1):
                idx = (lane + sh) & (_L - 1)
                acc = acc + jnp.take(acc, idx)
            out = jnp.where(lane == jj, acc, out)
        acc_v[pl.ds(g * _L, _L)] = out
        return carry

    lax.fori_loop(0, _RW // _L, group, 0)
    pltpu.sync_copy(acc_v, out_hbm.at[pl.ds(base, _RW)])


def _feat_sumsq(features):
    mesh = plsc.VectorSubcoreMesh(core_axis_name="c", subcore_axis_name="s")
    return pl.kernel(
        _feat_body,
        mesh=mesh,
        out_type=jax.ShapeDtypeStruct((_N,), jnp.float32),
        scratch_types=[
            pltpu.VMEM((_RW, _F), jnp.float32),
            pltpu.VMEM((_RW,), jnp.float32),
            pltpu.SemaphoreType.DMA,
        ],
    )(features)


def _dense(xt_ref, tgt_ref, loss_ref, unc_ref):
    x = xt_ref[...]
    t = tgt_ref[...]
    colmax = jnp.max(x, axis=0, keepdims=True)
    s = jnp.sum(jnp.exp(x - colmax), axis=0, keepdims=True)
    logs = jnp.log(s)
    rows = jax.lax.broadcasted_iota(jnp.int32, (_C, _B), 0)
    tl = jnp.sum(jnp.where(rows == t, x, 0.0), axis=0, keepdims=True)
    loss_ref[...] = (colmax + logs) - tl
    unc_ref[...] = 1.0 - 1.0 / s


def _select(loss_ref, unc_ref, fsq_ref, out_ref):
    l = loss_ref[...]
    u = unc_ref[...]
    fn = jnp.sqrt(fsq_ref[...])
    nf = jnp.float32(_N)
    mean = jnp.sum(l) / nf
    var = jnp.sum((l - mean) ** 2) / (nf - 1.0)
    std = jnp.sqrt(var)
    alpha = jnp.clip(_ALPHA * (1.0 + std), 0.05, 0.5)
    k = jnp.maximum(1, jnp.ceil(nf * alpha)).astype(jnp.int32)

    bits = jax.lax.bitcast_convert_type(l, jnp.int32)
    key = jnp.where(bits < 0, bits ^ jnp.int32(0x7FFFFFFF), bits)
    min32 = jnp.int32(-2147483648)

    def body_tau(j, t):
        t2 = t | (jnp.int32(1) << (jnp.int32(31) - j))
        c = jnp.sum((key >= (t2 ^ min32)).astype(jnp.int32))
        return jnp.where(c >= k, t2, t)

    tau_u = jax.lax.fori_loop(0, 32, body_tau, jnp.int32(0))
    tau = tau_u ^ min32

    c_gt = jnp.sum((key > tau).astype(jnp.int32))
    m = k - c_gt
    tied = key == tau
    ii = (jax.lax.broadcasted_iota(jnp.int32, (_R2, _R2), 0) * _R2
          + jax.lax.broadcasted_iota(jnp.int32, (_R2, _R2), 1))

    def body_idx(j, t):
        t2 = t | (jnp.int32(1) << (jnp.int32(13) - j))
        c = jnp.sum((tied & (ii < t2)).astype(jnp.int32))
        return jnp.where(c < m, t2, t)

    t_idx = jax.lax.fori_loop(0, 14, body_idx, jnp.int32(0))

    include = (key > tau) | (tied & (ii <= t_idx))
    contrib = l * (_BASE_MARGIN * (1.0 + _ADAPT_RATE * u)) + 0.1 * fn
    total = jnp.sum(jnp.where(include, contrib, 0.0))
    out_ref[...] = (total / k.astype(jnp.float32)).reshape(1, 1)


def kernel(outputs, targets, features):
    fsq = _feat_sumsq(features)
    xt = outputs.T
    tgt2 = targets.reshape(1, _N)
    loss, unc = pl.pallas_call(
        _dense,
        grid=(_NB,),
        in_specs=[
            pl.BlockSpec((_C, _B), lambda i: (0, i)),
            pl.BlockSpec((1, _B), lambda i: (0, i)),
        ],
        out_specs=[
            pl.BlockSpec((1, _B), lambda i: (0, i)),
            pl.BlockSpec((1, _B), lambda i: (0, i)),
        ],
        out_shape=[jax.ShapeDtypeStruct((1, _N), jnp.float32)] * 2,
    )(xt, tgt2)
    out = pl.pallas_call(
        _select,
        out_shape=jax.ShapeDtypeStruct((1, 1), jnp.float32),
    )(loss.reshape(_R2, _R2), unc.reshape(_R2, _R2), fsq.reshape(_R2, _R2))
    return out[0, 0]


# speedup vs baseline: 1.6669x; 1.4217x over previous
import jax
import jax.numpy as jnp
from jax.experimental import pallas as pl
from jax.experimental.pallas import tpu as pltpu

_ALPHA = 0.2
_BASE_MARGIN = 1.0
_ADAPT_RATE = 0.3

_N = 16384
_C = 1000
_F = 128
_B = 2048
_NB = _N // _B


def _body(xt_ref, tgt_ref, feat_ref, out_ref, loss_s, unc_s, fn_s):
    i = pl.program_id(0)

    @pl.when(i < _NB)
    def dense_step():
        x = xt_ref[...]
        t = tgt_ref[...]
        f = feat_ref[...]

        colmax = jnp.max(x, axis=0, keepdims=True)
        s = jnp.sum(jnp.exp(x - colmax), axis=0, keepdims=True)
        logs = jnp.log(s)
        rows = jax.lax.broadcasted_iota(jnp.int32, (_C, _B), 0)
        tl = jnp.sum(jnp.where(rows == t, x, 0.0), axis=0, keepdims=True)
        loss_s[pl.ds(i, 1), :] = (colmax + logs) - tl
        unc_s[pl.ds(i, 1), :] = 1.0 - 1.0 / s
        ones = jnp.ones((1, _F), dtype=jnp.float32)
        fsq = jax.lax.dot_general(
            ones, f * f, (((1,), (1,)), ((), ())),
            preferred_element_type=jnp.float32)
        fn_s[pl.ds(i, 1), :] = jnp.sqrt(fsq)

    @pl.when(i == _NB)
    def select_step():
        l = loss_s[...]
        u = unc_s[...]
        fn = fn_s[...]
        nf = jnp.float32(_N)
        mean = jnp.sum(l) / nf
        var = jnp.sum((l - mean) ** 2) / (nf - 1.0)
        std = jnp.sqrt(var)
        alpha = jnp.clip(_ALPHA * (1.0 + std), 0.05, 0.5)
        k = jnp.maximum(1, jnp.ceil(nf * alpha)).astype(jnp.int32)

        bits = jax.lax.bitcast_convert_type(l, jnp.int32)
        key = jnp.where(bits < 0, bits ^ jnp.int32(0x7FFFFFFF), bits)
        min32 = jnp.int32(-2147483648)

        def body_tau(j, t):
            t2 = t | (jnp.int32(1) << (jnp.int32(31) - j))
            c = jnp.sum((key >= (t2 ^ min32)).astype(jnp.int32))
            return jnp.where(c >= k, t2, t)

        tau_u = jax.lax.fori_loop(0, 32, body_tau, jnp.int32(0))
        tau = tau_u ^ min32

        c_gt = jnp.sum((key > tau).astype(jnp.int32))
        m = k - c_gt
        tied = key == tau
        ii = (jax.lax.broadcasted_iota(jnp.int32, (_NB, _B), 0) * _B
              + jax.lax.broadcasted_iota(jnp.int32, (_NB, _B), 1))

        def body_idx(j, t):
            t2 = t | (jnp.int32(1) << (jnp.int32(13) - j))
            c = jnp.sum((tied & (ii < t2)).astype(jnp.int32))
            return jnp.where(c < m, t2, t)

        t_idx = jax.lax.fori_loop(0, 14, body_idx, jnp.int32(0))

        include = (key > tau) | (tied & (ii <= t_idx))
        contrib = l * (_BASE_MARGIN * (1.0 + _ADAPT_RATE * u)) + 0.1 * fn
        total = jnp.sum(jnp.where(include, contrib, 0.0))
        out_ref[...] = (total / k.astype(jnp.float32)).reshape(1, 1)


def kernel(outputs, targets, features):
    xt = outputs.T
    tgt2 = targets.reshape(1, _N)
    last = _NB - 1
    out = pl.pallas_call(
        _body,
        grid=(_NB + 1,),
        in_specs=[
            pl.BlockSpec((_C, _B), lambda i: (0, jnp.minimum(i, last))),
            pl.BlockSpec((1, _B), lambda i: (0, jnp.minimum(i, last))),
            pl.BlockSpec((_B, _F), lambda i: (jnp.minimum(i, last), 0)),
        ],
        out_specs=pl.BlockSpec((1, 1), lambda i: (0, 0)),
        out_shape=jax.ShapeDtypeStruct((1, 1), jnp.float32),
        scratch_shapes=[pltpu.VMEM((_NB, _B), jnp.float32)] * 3,
    )(xt, tgt2, features)
    return out[0, 0]
